# baseline (speedup 1.0000x reference)
import functools

import jax
import jax.numpy as jnp
from jax import lax
from jax.experimental import pallas as pl
from jax.experimental.pallas import tpu as pltpu
from jax.experimental.pallas import tpu_sc as plsc

_B, _S, _D = 4, 8192, 1024
_NW = 32
_ROWS_PER_W = _S // _NW
_CHUNK = 32
_NCHUNK = _ROWS_PER_W // _CHUNK


_mesh = plsc.VectorSubcoreMesh(core_axis_name="c", subcore_axis_name="s")


@functools.partial(
    pl.kernel,
    mesh=_mesh,
    out_type=jax.ShapeDtypeStruct((_B, _S, _D), jnp.float32),
    scratch_types=[
        pltpu.VMEM((2, _CHUNK, _D), jnp.float32),
        pltpu.SemaphoreType.DMA,
        pltpu.SemaphoreType.DMA,
    ],
)
def _sc_broadcast(table_hbm, out_hbm, buf, in_sem, out_sem):
    wid = lax.axis_index("s") * 2 + lax.axis_index("c")
    base = wid * _ROWS_PER_W

    pending_out = []
    for c in range(_NCHUNK):
        r0 = base + c * _CHUNK
        slot = c % 2
        cp_in = pltpu.make_async_copy(
            table_hbm.at[pl.ds(r0, _CHUNK)], buf.at[slot], in_sem
        )
        cp_in.start()
        for cp in pending_out:
            cp.wait()
        pending_out = []
        cp_in.wait()
        for b in range(_B):
            cp_out = pltpu.make_async_copy(
                buf.at[slot], out_hbm.at[b, pl.ds(r0, _CHUNK)], out_sem
            )
            cp_out.start()
            pending_out.append(cp_out)
    for cp in pending_out:
        cp.wait()


def kernel(x, table):
    return _sc_broadcast(table)


# speedup vs baseline: 1.4695x; 1.4695x over previous
import jax
import jax.numpy as jnp
from jax.experimental import pallas as pl


_BLOCK_S = 1024


def _copy_kernel(table_ref, out_ref):
    out_ref[...] = jnp.broadcast_to(table_ref[...][None], out_ref.shape)


def kernel(x, table):
    B, S, D = x.shape
    grid = (S // _BLOCK_S,)
    return pl.pallas_call(
        _copy_kernel,
        grid=grid,
        in_specs=[
            pl.BlockSpec((_BLOCK_S, D), lambda s: (s, 0)),
        ],
        out_specs=pl.BlockSpec((B, _BLOCK_S, D), lambda s: (0, s, 0)),
        out_shape=jax.ShapeDtypeStruct((B, S, D), table.dtype),
    )(table[:S])
